# Initial kernel scaffold; baseline (speedup 1.0000x reference)
import functools

import jax
import jax.numpy as jnp
from jax import lax
from jax.experimental import pallas as pl
from jax.experimental.pallas import tpu as pltpu
from jax.experimental.pallas import tpu_sc as plsc

NUM_EMB = 100000
SEM_IDS_DIM = 4
EMB_DIM = 64
N_SEM = 3
MAX_TAG = 1000
N_TAG = SEM_IDS_DIM - N_SEM
SEM_OFF = NUM_EMB * N_SEM
TOTAL_EMB = SEM_OFF + MAX_TAG * N_TAG + 1
PAD_IDX = TOTAL_EMB - 1
B, L = 4096, 200
LF = 4

NC = 2
NS = 16
NW = NC * NS
LANES = 16

CHUNK = 512
IDX_ROWS = CHUNK // 128

SEQ_N = B * L
FUT_N = B * LF
SEQ_PER_W = SEQ_N // NW
FUT_PER_W = FUT_N // NW
SEQ_CHUNKS = SEQ_PER_W // CHUNK


def _compute_indices_chunk(sem_v, tok_v, idx_v):
    for i in range(CHUNK // LANES):
        s = sem_v[pl.ds(i * LANES, LANES)]
        t = tok_v[pl.ds(i * LANES, LANES)]
        sem_c = jnp.minimum(jnp.maximum(s, 0), NUM_EMB - 1)
        tag_c = jnp.minimum(jnp.maximum(s, 0), MAX_TAG - 1)
        idx_sem = t * NUM_EMB + sem_c
        tag_layer = t - N_SEM
        idx_tag = jnp.where(
            tag_layer < N_TAG, SEM_OFF + tag_layer * MAX_TAG + tag_c, PAD_IDX
        )
        idx = jnp.where(t < N_SEM, idx_sem, idx_tag)
        idx_v[i // 8, pl.ds((i % 8) * LANES, LANES)] = idx


def _gather_chunk(sem_hbm, tok_hbm, table_hbm, out_hbm, base,
                  sem_v, tok_v, idx_v, rows_v, dsem):
    pltpu.sync_copy(sem_hbm.at[pl.ds(base, CHUNK)], sem_v)
    pltpu.sync_copy(tok_hbm.at[pl.ds(base, CHUNK)], tok_v)
    _compute_indices_chunk(sem_v, tok_v, idx_v)
    copies = [
        pltpu.async_copy(
            table_hbm.at[idx_v.at[j]],
            rows_v.at[pl.ds(j * 128, 128)],
            dsem,
        )
        for j in range(IDX_ROWS)
    ]
    for c in copies:
        c.wait()
    pltpu.sync_copy(rows_v, out_hbm.at[pl.ds(base, CHUNK)])


def _body(sem_seq, tok_seq, sem_fut, tok_fut, table,
          out_seq, out_fut, sem_v, tok_v, idx_v, rows_v, dsem):
    wid = lax.axis_index("s") * NC + lax.axis_index("c")
    seq_base0 = wid * SEQ_PER_W

    def chunk_fn(i, carry):
        base = seq_base0 + i * CHUNK
        _gather_chunk(sem_seq, tok_seq, table, out_seq, base,
                      sem_v, tok_v, idx_v, rows_v, dsem)
        return carry

    lax.fori_loop(0, SEQ_CHUNKS, chunk_fn, 0)

    fut_base = wid * FUT_PER_W
    _gather_chunk(sem_fut, tok_fut, table, out_fut, fut_base,
                  sem_v, tok_v, idx_v, rows_v, dsem)


@jax.jit
def _emb_lookup(sem_seq, tok_seq, sem_fut, tok_fut, table):
    mesh = plsc.VectorSubcoreMesh(core_axis_name="c", subcore_axis_name="s")
    f = pl.kernel(
        _body,
        out_type=(
            jax.ShapeDtypeStruct((SEQ_N, EMB_DIM), jnp.float32),
            jax.ShapeDtypeStruct((FUT_N, EMB_DIM), jnp.float32),
        ),
        mesh=mesh,
        scratch_types=[
            pltpu.VMEM((CHUNK,), jnp.int32),
            pltpu.VMEM((CHUNK,), jnp.int32),
            pltpu.VMEM((IDX_ROWS, 128), jnp.int32),
            pltpu.VMEM((CHUNK, EMB_DIM), jnp.float32),
            pltpu.SemaphoreType.DMA,
        ],
    )
    return f(sem_seq, tok_seq, sem_fut, tok_fut, table)


def kernel(sem_ids, token_type_ids, sem_ids_fut, token_type_ids_fut, emb_table):
    out_seq, out_fut = _emb_lookup(
        sem_ids.reshape(-1),
        token_type_ids.reshape(-1),
        sem_ids_fut.reshape(-1),
        token_type_ids_fut.reshape(-1),
        emb_table,
    )
    return (
        out_seq.reshape(B, L, EMB_DIM),
        out_fut.reshape(B, LF, EMB_DIM),
    )


# speedup vs baseline: 1.8005x; 1.8005x over previous
import functools

import jax
import jax.numpy as jnp
from jax import lax
from jax.experimental import pallas as pl
from jax.experimental.pallas import tpu as pltpu
from jax.experimental.pallas import tpu_sc as plsc

NUM_EMB = 100000
SEM_IDS_DIM = 4
EMB_DIM = 64
N_SEM = 3
MAX_TAG = 1000
N_TAG = SEM_IDS_DIM - N_SEM
SEM_OFF = NUM_EMB * N_SEM
TOTAL_EMB = SEM_OFF + MAX_TAG * N_TAG + 1
PAD_IDX = TOTAL_EMB - 1
B, L = 4096, 200
LF = 4

NC = 2
NS = 16
NW = NC * NS
LANES = 16

CHUNK = 512
IDX_ROWS = CHUNK // 128

SEQ_N = B * L
FUT_N = B * LF
SEQ_PER_W = SEQ_N // NW
FUT_PER_W = FUT_N // NW
SEQ_CHUNKS = SEQ_PER_W // CHUNK


def _compute_indices_chunk(sem_v, tok_v, idx_v):
    for i in range(CHUNK // LANES):
        s = sem_v[pl.ds(i * LANES, LANES)]
        t = tok_v[pl.ds(i * LANES, LANES)]
        sem_c = jnp.minimum(jnp.maximum(s, 0), NUM_EMB - 1)
        tag_c = jnp.minimum(jnp.maximum(s, 0), MAX_TAG - 1)
        idx_sem = t * NUM_EMB + sem_c
        tag_layer = t - N_SEM
        idx_tag = jnp.where(
            tag_layer < N_TAG, SEM_OFF + tag_layer * MAX_TAG + tag_c, PAD_IDX
        )
        idx = jnp.where(t < N_SEM, idx_sem, idx_tag)
        idx_v[i // 8, pl.ds((i % 8) * LANES, LANES)] = idx


def _gather_chunk(sem_hbm, tok_hbm, table_hbm, out_hbm, base,
                  sem_v, tok_v, idx_v, rows_v, dsem):
    pltpu.sync_copy(sem_hbm.at[pl.ds(base, CHUNK)], sem_v)
    pltpu.sync_copy(tok_hbm.at[pl.ds(base, CHUNK)], tok_v)
    _compute_indices_chunk(sem_v, tok_v, idx_v)
    copies = [
        pltpu.async_copy(
            table_hbm.at[idx_v.at[j]],
            rows_v.at[pl.ds(j * 128, 128)],
            dsem,
        )
        for j in range(IDX_ROWS)
    ]
    for c in copies:
        c.wait()
    pltpu.sync_copy(rows_v, out_hbm.at[pl.ds(base, CHUNK)])


def _body(sem_seq, tok_seq, sem_fut, tok_fut, table,
          out_seq, out_fut, sem_v, tok_v, idx_v, rows_v, dsem):
    wid = lax.axis_index("s") * NC + lax.axis_index("c")
    seq_base0 = wid * SEQ_PER_W

    def chunk_fn(i, carry):
        base = seq_base0 + i * CHUNK
        _gather_chunk(sem_seq, tok_seq, table, out_seq, base,
                      sem_v, tok_v, idx_v, rows_v, dsem)
        return carry

    lax.fori_loop(0, SEQ_CHUNKS, chunk_fn, 0)

    fut_base = wid * FUT_PER_W
    _gather_chunk(sem_fut, tok_fut, table, out_fut, fut_base,
                  sem_v, tok_v, idx_v, rows_v, dsem)


@jax.jit
def _emb_lookup(sem_seq, tok_seq, sem_fut, tok_fut, table):
    mesh = plsc.VectorSubcoreMesh(core_axis_name="c", subcore_axis_name="s")
    f = pl.kernel(
        _body,
        out_type=(
            jax.ShapeDtypeStruct((SEQ_N, EMB_DIM), jnp.float32),
            jax.ShapeDtypeStruct((FUT_N, EMB_DIM), jnp.float32),
        ),
        mesh=mesh,
        scratch_types=[
            pltpu.VMEM((CHUNK,), jnp.int32),
            pltpu.VMEM((CHUNK,), jnp.int32),
            pltpu.VMEM((IDX_ROWS, 128), jnp.int32),
            pltpu.VMEM((CHUNK, EMB_DIM), jnp.float32),
            pltpu.SemaphoreType.DMA,
        ],
        compiler_params=pltpu.CompilerParams(use_tc_tiling_on_sc=False),
    )
    return f(sem_seq, tok_seq, sem_fut, tok_fut, table)


def kernel(sem_ids, token_type_ids, sem_ids_fut, token_type_ids_fut, emb_table):
    out_seq, out_fut = _emb_lookup(
        sem_ids.reshape(-1),
        token_type_ids.reshape(-1),
        sem_ids_fut.reshape(-1),
        token_type_ids_fut.reshape(-1),
        emb_table,
    )
    return (
        out_seq.reshape(B, L, EMB_DIM),
        out_fut.reshape(B, LF, EMB_DIM),
    )
